# Initial kernel scaffold; baseline (speedup 1.0000x reference)
import jax
import jax.numpy as jnp
from jax.experimental import pallas as pl


def kernel(x, table, W1, b1, W2, b2):
    raise NotImplementedError("write your pallas kernel here")


# speedup vs baseline: 25.1321x; 25.1321x over previous
import functools

import jax
import jax.numpy as jnp
from jax import lax
from jax.experimental import pallas as pl
from jax.experimental.pallas import tpu as pltpu
from jax.experimental.pallas import tpu_sc as plsc

B = 16384
S = 200
D = 32
HID = 128
OUT = 64

NC = 2
NS = 16
NW = NC * NS
BPW = B // NW
G = 4
NG = BPW // G
IDX_PER_GROUP = G * S
NCHUNK = IDX_PER_GROUP // 100


def _pool_body(x_hbm, table_hbm, out_hbm, idx_v, rows_v, out_v, sem):
    wid = lax.axis_index("s") * NC + lax.axis_index("c")
    base = wid * BPW

    def group(g, carry):
        pltpu.sync_copy(x_hbm.at[wid, g], idx_v)
        for j in range(NCHUNK):
            pltpu.async_copy(table_hbm.at[idx_v.at[j]], rows_v.at[j], sem)
        for j in range(NCHUNK):
            pltpu.make_async_copy(table_hbm.at[idx_v.at[j]], rows_v.at[j], sem).wait()
        for i in range(G):
            acc0 = jnp.zeros((16,), jnp.float32)
            acc1 = jnp.zeros((16,), jnp.float32)
            for j2 in range(2):
                c = 2 * i + j2

                def rbody(r, accs, c=c):
                    a0, a1 = accs
                    for u in range(10):
                        rr = r * 10 + u
                        a0 = a0 + rows_v[c, rr, 0:16]
                        a1 = a1 + rows_v[c, rr, 16:32]
                    return a0, a1

                acc0, acc1 = lax.fori_loop(0, 10, rbody, (acc0, acc1))
            b_loc = g * G + i
            out_v[b_loc, 0:16] = acc0
            out_v[b_loc, 16:32] = acc1
        return carry

    lax.fori_loop(0, NG, group, 0)
    pltpu.sync_copy(out_v, out_hbm.at[pl.ds(base, BPW)])


@jax.jit
def _pool(x4, table):
    mesh = plsc.VectorSubcoreMesh(core_axis_name="c", subcore_axis_name="s")
    return pl.kernel(
        _pool_body,
        out_type=jax.ShapeDtypeStruct((B, D), jnp.float32),
        mesh=mesh,
        scratch_types=[
            pltpu.VMEM((NCHUNK, 100), jnp.int32),
            pltpu.VMEM((NCHUNK, 100, D), jnp.float32),
            pltpu.VMEM((BPW, D), jnp.float32),
            pltpu.SemaphoreType.DMA,
        ],
        compiler_params=pltpu.CompilerParams(use_tc_tiling_on_sc=False),
    )(x4, table)


def _mlp_body(p_ref, w1_ref, b1_ref, w2_ref, b2_ref, o_ref):
    h = jnp.dot(p_ref[...], w1_ref[...], preferred_element_type=jnp.float32)
    h = jnp.maximum(h + b1_ref[...], 0.0)
    o = jnp.dot(h, w2_ref[...], preferred_element_type=jnp.float32)
    o_ref[...] = o + b2_ref[...]


@jax.jit
def _mlp(pooled, W1t, b1, W2t, b2):
    BT = 1024
    return pl.pallas_call(
        _mlp_body,
        grid=(B // BT,),
        in_specs=[
            pl.BlockSpec((BT, D), lambda i: (i, 0)),
            pl.BlockSpec((D, HID), lambda i: (0, 0)),
            pl.BlockSpec((1, HID), lambda i: (0, 0)),
            pl.BlockSpec((HID, OUT), lambda i: (0, 0)),
            pl.BlockSpec((1, OUT), lambda i: (0, 0)),
        ],
        out_specs=pl.BlockSpec((BT, OUT), lambda i: (i, 0)),
        out_shape=jax.ShapeDtypeStruct((B, OUT), jnp.float32),
    )(pooled, W1t, b1, W2t, b2)


def kernel(x, table, W1, b1, W2, b2):
    x4 = x.astype(jnp.int32).reshape(NW, NG, NCHUNK, 100)
    sums = _pool(x4, table)
    W1t = W1.T * jnp.float32(1.0 / S)
    return _mlp(sums, W1t, b1.reshape(1, HID), W2.T, b2.reshape(1, OUT))


# speedup vs baseline: 37.2382x; 1.4817x over previous
import functools

import jax
import jax.numpy as jnp
from jax import lax
from jax.experimental import pallas as pl
from jax.experimental.pallas import tpu as pltpu
from jax.experimental.pallas import tpu_sc as plsc

B = 16384
S = 200
D = 32
HID = 128
OUT = 64

NC = 2
NS = 16
NW = NC * NS
BPW = B // NW
G = 4
NG = BPW // G
IDX_PER_GROUP = G * S
NCHUNK = IDX_PER_GROUP // 100


def _pool_body(x_hbm, table_hbm, out_hbm, idx_v, rows_v, out_v, idx_sem, gat_sem):
    wid = lax.axis_index("s") * NC + lax.axis_index("c")
    base = wid * BPW

    def start_gathers(slot):
        for j in range(NCHUNK):
            pltpu.async_copy(table_hbm.at[idx_v.at[slot, j]], rows_v.at[slot, j], gat_sem)

    def drain_gathers(slot):
        for j in range(NCHUNK):
            pltpu.make_async_copy(
                table_hbm.at[idx_v.at[slot, j]], rows_v.at[slot, j], gat_sem
            ).wait()

    def wait_idx(slot):
        pltpu.make_async_copy(x_hbm.at[wid, 0], idx_v.at[slot], idx_sem).wait()

    pltpu.async_copy(x_hbm.at[wid, 0], idx_v.at[0], idx_sem)
    pltpu.async_copy(x_hbm.at[wid, 1], idx_v.at[1], idx_sem)
    wait_idx(0)
    start_gathers(0)

    def group(g, carry):
        buf = g % 2
        nb = 1 - buf
        drain_gathers(buf)

        @pl.when(g + 2 < NG)
        def _():
            gg = jnp.minimum(g + 2, NG - 1)
            pltpu.async_copy(x_hbm.at[wid, gg], idx_v.at[buf], idx_sem)

        @pl.when(g + 1 < NG)
        def _():
            wait_idx(nb)
            start_gathers(nb)

        for i in range(G):
            acc0 = jnp.zeros((16,), jnp.float32)
            acc1 = jnp.zeros((16,), jnp.float32)
            for j2 in range(2):
                c = 2 * i + j2

                def rbody(r, accs, c=c):
                    a0, a1 = accs
                    for u in range(10):
                        rr = r * 10 + u
                        a0 = a0 + rows_v[buf, c, rr, 0:16]
                        a1 = a1 + rows_v[buf, c, rr, 16:32]
                    return a0, a1

                acc0, acc1 = lax.fori_loop(0, 10, rbody, (acc0, acc1))
            b_loc = g * G + i
            out_v[b_loc, 0:16] = acc0
            out_v[b_loc, 16:32] = acc1
        return carry

    lax.fori_loop(0, NG, group, 0)
    pltpu.sync_copy(out_v, out_hbm.at[pl.ds(base, BPW)])


@jax.jit
def _pool(x4, table):
    mesh = plsc.VectorSubcoreMesh(core_axis_name="c", subcore_axis_name="s")
    return pl.kernel(
        _pool_body,
        out_type=jax.ShapeDtypeStruct((B, D), jnp.float32),
        mesh=mesh,
        scratch_types=[
            pltpu.VMEM((2, NCHUNK, 100), jnp.int32),
            pltpu.VMEM((2, NCHUNK, 100, D), jnp.float32),
            pltpu.VMEM((BPW, D), jnp.float32),
            pltpu.SemaphoreType.DMA,
            pltpu.SemaphoreType.DMA,
        ],
        compiler_params=pltpu.CompilerParams(use_tc_tiling_on_sc=False),
    )(x4, table)


def _mlp_body(p_ref, w1_ref, b1_ref, w2_ref, b2_ref, o_ref):
    h = jnp.dot(p_ref[...], w1_ref[...], preferred_element_type=jnp.float32)
    h = jnp.maximum(h + b1_ref[...], 0.0)
    o = jnp.dot(h, w2_ref[...], preferred_element_type=jnp.float32)
    o_ref[...] = o + b2_ref[...]


@jax.jit
def _mlp(pooled, W1t, b1, W2t, b2):
    BT = 1024
    return pl.pallas_call(
        _mlp_body,
        grid=(B // BT,),
        in_specs=[
            pl.BlockSpec((BT, D), lambda i: (i, 0)),
            pl.BlockSpec((D, HID), lambda i: (0, 0)),
            pl.BlockSpec((1, HID), lambda i: (0, 0)),
            pl.BlockSpec((HID, OUT), lambda i: (0, 0)),
            pl.BlockSpec((1, OUT), lambda i: (0, 0)),
        ],
        out_specs=pl.BlockSpec((BT, OUT), lambda i: (i, 0)),
        out_shape=jax.ShapeDtypeStruct((B, OUT), jnp.float32),
    )(pooled, W1t, b1, W2t, b2)


def kernel(x, table, W1, b1, W2, b2):
    x4 = x.astype(jnp.int32).reshape(NW, NG, NCHUNK, 100)
    sums = _pool(x4, table)
    W1t = W1.T * jnp.float32(1.0 / S)
    return _mlp(sums, W1t, b1.reshape(1, HID), W2.T, b2.reshape(1, OUT))


# speedup vs baseline: 40.2945x; 1.0821x over previous
import functools

import jax
import jax.numpy as jnp
from jax import lax
from jax.experimental import pallas as pl
from jax.experimental.pallas import tpu as pltpu
from jax.experimental.pallas import tpu_sc as plsc

B = 16384
S = 200
D = 32
HID = 128
OUT = 64

NC = 2
NS = 16
NW = NC * NS
BPW = B // NW
G = 4
NG = BPW // G
IDX_PER_GROUP = G * S
NCHUNK = IDX_PER_GROUP // 100


def _pool_body(x_hbm, table_hbm, out_hbm, idx_v, rows_v, out_v, idx_sem, gat_sem):
    wid = lax.axis_index("s") * NC + lax.axis_index("c")
    base = wid * BPW

    SPLITS = ((0, 96), (96, 104))

    def start_gathers(slot):
        for i in range(G):
            for off, sz in SPLITS:
                pltpu.async_copy(
                    table_hbm.at[idx_v.at[slot, i, pl.ds(off, sz)]],
                    rows_v.at[slot, i, pl.ds(off, sz)],
                    gat_sem,
                )

    def drain_gathers(slot):
        for i in range(G):
            for off, sz in SPLITS:
                pltpu.make_async_copy(
                    table_hbm.at[idx_v.at[slot, i, pl.ds(off, sz)]],
                    rows_v.at[slot, i, pl.ds(off, sz)],
                    gat_sem,
                ).wait()

    def start_idx(g, slot):
        pltpu.async_copy(x_hbm.at[pl.ds(base + g * G, G)], idx_v.at[slot], idx_sem)

    def wait_idx(slot):
        pltpu.make_async_copy(x_hbm.at[pl.ds(0, G)], idx_v.at[slot], idx_sem).wait()

    start_idx(0, 0)
    start_idx(1, 1)
    wait_idx(0)
    start_gathers(0)

    def group(g, carry):
        buf = g % 2
        nb = 1 - buf
        drain_gathers(buf)

        @pl.when(g + 2 < NG)
        def _():
            gg = jnp.minimum(g + 2, NG - 1)
            start_idx(gg, buf)

        @pl.when(g + 1 < NG)
        def _():
            wait_idx(nb)
            start_gathers(nb)

        for i in range(G):

            def rbody(r, accs, i=i):
                a0, a1 = accs
                for u in range(10):
                    rr = r * 10 + u
                    a0 = a0 + rows_v[buf, i, rr, 0:16]
                    a1 = a1 + rows_v[buf, i, rr, 16:32]
                return a0, a1

            acc0, acc1 = lax.fori_loop(
                0, 20, rbody, (jnp.zeros((16,), jnp.float32), jnp.zeros((16,), jnp.float32))
            )
            b_loc = g * G + i
            out_v[b_loc, 0:16] = acc0
            out_v[b_loc, 16:32] = acc1
        return carry

    lax.fori_loop(0, NG, group, 0)
    pltpu.sync_copy(out_v, out_hbm.at[pl.ds(base, BPW)])


@jax.jit
def _pool(x2, table):
    mesh = plsc.VectorSubcoreMesh(core_axis_name="c", subcore_axis_name="s")
    return pl.kernel(
        _pool_body,
        out_type=jax.ShapeDtypeStruct((B, D), jnp.float32),
        mesh=mesh,
        scratch_types=[
            pltpu.VMEM((2, G, S), jnp.int32),
            pltpu.VMEM((2, G, S, D), jnp.float32),
            pltpu.VMEM((BPW, D), jnp.float32),
            pltpu.SemaphoreType.DMA,
            pltpu.SemaphoreType.DMA,
        ],
        compiler_params=pltpu.CompilerParams(use_tc_tiling_on_sc=False),
    )(x2, table)


def _mlp_body(p_ref, w1_ref, b1_ref, w2_ref, b2_ref, o_ref):
    h = jnp.dot(p_ref[...], w1_ref[...], preferred_element_type=jnp.float32)
    h = jnp.maximum(h + b1_ref[...], 0.0)
    o = jnp.dot(h, w2_ref[...], preferred_element_type=jnp.float32)
    o_ref[...] = o + b2_ref[...]


@jax.jit
def _mlp(pooled, W1t, b1, W2t, b2):
    BT = 1024
    return pl.pallas_call(
        _mlp_body,
        grid=(B // BT,),
        in_specs=[
            pl.BlockSpec((BT, D), lambda i: (i, 0)),
            pl.BlockSpec((D, HID), lambda i: (0, 0)),
            pl.BlockSpec((1, HID), lambda i: (0, 0)),
            pl.BlockSpec((HID, OUT), lambda i: (0, 0)),
            pl.BlockSpec((1, OUT), lambda i: (0, 0)),
        ],
        out_specs=pl.BlockSpec((BT, OUT), lambda i: (i, 0)),
        out_shape=jax.ShapeDtypeStruct((B, OUT), jnp.float32),
    )(pooled, W1t, b1, W2t, b2)


def kernel(x, table, W1, b1, W2, b2):
    sums = _pool(x.astype(jnp.int32), table)
    W1t = W1.T * jnp.float32(1.0 / S)
    return _mlp(sums, W1t, b1.reshape(1, HID), W2.T, b2.reshape(1, OUT))
